# baseline (device time: 14512 ns/iter reference)
import jax
import jax.numpy as jnp
from jax import lax
from jax.experimental import pallas as pl
from jax.experimental.pallas import tpu as pltpu

M_BLK = 256


def kernel(x, dy, gamma):
    m, d = x.shape
    m_half = m // 2
    n_blk = m_half // M_BLK

    my_x = lax.axis_index("x")
    off = (my_x * n_blk).astype(jnp.int32).reshape((1,))

    def body(off_ref, x_ref, dy_ref, out_ref, acc_ref, rbuf_ref,
             send_sems, recv_sems):
        i = pl.program_id(0)
        mx = lax.axis_index("x")
        my = lax.axis_index("y")
        peers = [(1 - mx, my), (mx, 1 - my), (1 - mx, 1 - my)]

        @pl.when(i == 0)
        def _():
            barrier_sem = pltpu.get_barrier_semaphore()
            for nbr in peers:
                pl.semaphore_signal(
                    barrier_sem, inc=1, device_id=nbr,
                    device_id_type=pl.DeviceIdType.MESH,
                )
            pl.semaphore_wait(barrier_sem, 3)

        xv = x_ref[:, :]
        dyv = dy_ref[:, :]
        mu = jnp.mean(xv, axis=1, keepdims=True)
        xc = xv - mu
        var = jnp.mean(xc * xc, axis=1, keepdims=True)
        xhat = xc * lax.rsqrt(var + 1e-5)
        dgamma = jnp.sum(dyv * xhat, axis=0, keepdims=True)
        dbeta = jnp.sum(dyv, axis=0, keepdims=True)
        blk = jnp.concatenate([dgamma, dbeta], axis=0)

        @pl.when(i == 0)
        def _():
            acc_ref[:, :] = blk

        @pl.when(i > 0)
        def _():
            acc_ref[:, :] = acc_ref[:, :] + blk

        @pl.when(i == n_blk - 1)
        def _():
            rdmas = []
            for k, nbr in enumerate(peers):
                rdma = pltpu.make_async_remote_copy(
                    src_ref=acc_ref,
                    dst_ref=rbuf_ref.at[k],
                    send_sem=send_sems.at[k],
                    recv_sem=recv_sems.at[k],
                    device_id=nbr,
                    device_id_type=pl.DeviceIdType.MESH,
                )
                rdma.start()
                rdmas.append(rdma)
            rdmas[0].wait()
            s01 = acc_ref[:, :] + rbuf_ref[0, :, :]
            rdmas[1].wait()
            s01 = s01 + rbuf_ref[1, :, :]
            rdmas[2].wait()
            out_ref[:, :] = s01 + rbuf_ref[2, :, :]

    grid_spec = pltpu.PrefetchScalarGridSpec(
        num_scalar_prefetch=1,
        grid=(n_blk,),
        in_specs=[
            pl.BlockSpec((M_BLK, d), lambda i, off: (off[0] + i, 0)),
            pl.BlockSpec((M_BLK, d), lambda i, off: (off[0] + i, 0)),
        ],
        out_specs=pl.BlockSpec((2, d), lambda i, off: (0, 0)),
        scratch_shapes=[
            pltpu.VMEM((2, d), jnp.float32),
            pltpu.VMEM((3, 2, d), jnp.float32),
            pltpu.SemaphoreType.DMA((3,)),
            pltpu.SemaphoreType.DMA((3,)),
        ],
    )

    return pl.pallas_call(
        body,
        grid_spec=grid_spec,
        out_shape=jax.ShapeDtypeStruct((2, d), jnp.float32),
        compiler_params=pltpu.CompilerParams(
            collective_id=0,
            vmem_limit_bytes=120 * 1024 * 1024,
        ),
    )(off, x, dy)
